# baseline (speedup 1.0000x reference)
import functools

import jax
import jax.numpy as jnp
from jax import lax
from jax.experimental import pallas as pl
from jax.experimental.pallas import tpu as pltpu
from jax.experimental.pallas import tpu_sc as plsc

N = 10000
E = 160000
F = 16
ED = 4
EW = 256
G = 64

NC = 2
NS = 16
L = 16
NW = NC * NS
CH = 125
CHP = 128
NCHUNK = (E // NW) // CH
WPE = NCHUNK * CHP
EP = NW * WPE
NP = N + L
NPS = NP // NS
NWS = N // NS
BE = 4096

@functools.cache
def _sc_mesh():
    return plsc.VectorSubcoreMesh(
        core_axis_name="c", subcore_axis_name="s",
        num_cores=NC, num_subcores=NS)


def _zero_acc(zbuf, acc, sid):
    def zrow(i, c):
        zbuf[i, :] = jnp.zeros((L,), jnp.float32)
        return c
    lax.fori_loop(0, NPS, zrow, 0)
    pltpu.sync_copy(zbuf, acc.at[pl.ds(sid * NPS, NPS)])
    plsc.subcore_barrier()


def _writeout_acc(acc, out, cid, sid):
    plsc.subcore_barrier()
    @pl.when(sid == 0)
    def _():
        pltpu.sync_copy(acc.at[pl.ds(0, N)], out.at[cid])


@functools.cache
def _sc_gather_k():
    @functools.partial(
        pl.kernel,
        out_type=jax.ShapeDtypeStruct((EP, F), jnp.float32),
        mesh=_sc_mesh(),
        compiler_params=pltpu.CompilerParams(use_tc_tiling_on_sc=False),
        scratch_types=[
            pltpu.VMEM((NCHUNK, CHP), jnp.int32),
            pltpu.VMEM((WPE, F), jnp.float32),
            pltpu.VMEM_SHARED((N, F), jnp.float32),
            pltpu.SemaphoreType.DMA,
        ],
    )
    def k(table, idx, out, idx_v, rows_v, tsh, sem):
        cid = lax.axis_index("c")
        sid = lax.axis_index("s")
        wid = sid * NC + cid
        pltpu.sync_copy(idx.at[pl.ds(wid * NCHUNK, NCHUNK)], idx_v)

        @pl.when(sid == 0)
        def _():
            pltpu.sync_copy(table, tsh)
        plsc.subcore_barrier()

        def fire(j, c):
            pltpu.make_async_copy(tsh.at[idx_v.at[j]],
                                  rows_v.at[pl.ds(j * CHP, CHP)], sem).start()
            return c
        lax.fori_loop(0, NCHUNK, fire, 0)

        def drain(j, c):
            pltpu.make_async_copy(tsh.at[idx_v.at[j]],
                                  rows_v.at[pl.ds(j * CHP, CHP)], sem).wait()
            return c
        lax.fori_loop(0, NCHUNK, drain, 0)
        pltpu.sync_copy(rows_v, out.at[pl.ds(wid * WPE, WPE)])
    return k


def _sc_gather(table, idxp):
    return _sc_gather_k()(table, idxp)


@functools.cache
def _sc_gather_count_k():
    @functools.partial(
        pl.kernel,
        out_type=(jax.ShapeDtypeStruct((EP, F), jnp.float32),
                  jax.ShapeDtypeStruct((NC, N, F), jnp.float32)),
        mesh=_sc_mesh(),
        compiler_params=pltpu.CompilerParams(use_tc_tiling_on_sc=False),
        scratch_types=[
            pltpu.VMEM((NCHUNK, CHP), jnp.int32),
            pltpu.VMEM((NCHUNK, CHP), jnp.int32),
            pltpu.VMEM((WPE, F), jnp.float32),
            pltpu.VMEM((CHP, F), jnp.float32),
            pltpu.VMEM((NPS, F), jnp.float32),
            pltpu.VMEM_SHARED((N, F), jnp.float32),
            pltpu.VMEM_SHARED((NP, F), jnp.float32),
            pltpu.SemaphoreType.DMA,
            pltpu.SemaphoreType.DMA,
        ],
    )
    def k(table, sidx, didx, out, cnt, sidx_v, didx_v, rows_v, ones_v, zbuf,
          tsh, acc, gsem, csem):
        cid = lax.axis_index("c")
        sid = lax.axis_index("s")
        wid = sid * NC + cid
        pltpu.sync_copy(sidx.at[pl.ds(wid * NCHUNK, NCHUNK)], sidx_v)
        pltpu.sync_copy(didx.at[pl.ds(wid * NCHUNK, NCHUNK)], didx_v)

        def orow(i, c):
            ones_v[i, :] = jnp.ones((L,), jnp.float32)
            return c
        lax.fori_loop(0, CHP, orow, 0)

        @pl.when(sid == 0)
        def _():
            pltpu.sync_copy(table, tsh)
        _zero_acc(zbuf, acc, sid)

        def gfire(j, c):
            pltpu.make_async_copy(tsh.at[sidx_v.at[j]],
                                  rows_v.at[pl.ds(j * CHP, CHP)], gsem).start()
            return c
        lax.fori_loop(0, NCHUNK, gfire, 0)

        def cfire(j, c):
            pltpu.async_copy(ones_v, acc.at[didx_v.at[j]], csem, add=True)
            return c
        lax.fori_loop(0, NCHUNK, cfire, 0)

        def gdrain(j, c):
            pltpu.make_async_copy(tsh.at[sidx_v.at[j]],
                                  rows_v.at[pl.ds(j * CHP, CHP)], gsem).wait()
            return c
        lax.fori_loop(0, NCHUNK, gdrain, 0)
        pltpu.sync_copy(rows_v, out.at[pl.ds(wid * WPE, WPE)])

        def cdrain(j, c):
            pltpu.make_async_copy(ones_v, acc.at[didx_v.at[j]], csem).wait()
            return c
        lax.fori_loop(0, NCHUNK, cdrain, 0)
        _writeout_acc(acc, cnt, cid, sid)
    return k


def _sc_gather_count(table, sidxp, didxp):
    return _sc_gather_count_k()(table, sidxp, didxp)


@functools.cache
def _sc_scatter_k():
    @functools.partial(
        pl.kernel,
        out_type=jax.ShapeDtypeStruct((NC, N, F), jnp.float32),
        mesh=_sc_mesh(),
        compiler_params=pltpu.CompilerParams(use_tc_tiling_on_sc=False),
        scratch_types=[
            pltpu.VMEM((NCHUNK, CHP), jnp.int32),
            pltpu.VMEM((WPE, F), jnp.float32),
            pltpu.VMEM((NPS, F), jnp.float32),
            pltpu.VMEM_SHARED((NP, F), jnp.float32),
            pltpu.SemaphoreType.DMA,
        ],
    )
    def k(msg, idx, out, idx_v, rows_v, zbuf, acc, sem):
        cid = lax.axis_index("c")
        sid = lax.axis_index("s")
        wid = sid * NC + cid
        pltpu.sync_copy(idx.at[pl.ds(wid * NCHUNK, NCHUNK)], idx_v)
        cp = pltpu.make_async_copy(msg.at[pl.ds(wid * WPE, WPE)], rows_v, sem)
        cp.start()
        _zero_acc(zbuf, acc, sid)
        cp.wait()

        def fire(j, c):
            pltpu.async_copy(rows_v.at[pl.ds(j * CHP, CHP)],
                             acc.at[idx_v.at[j]], sem, add=True)
            return c
        lax.fori_loop(0, NCHUNK, fire, 0)

        def drain(j, c):
            pltpu.make_async_copy(rows_v.at[pl.ds(j * CHP, CHP)],
                                  acc.at[idx_v.at[j]], sem).wait()
            return c
        lax.fori_loop(0, NCHUNK, drain, 0)
        _writeout_acc(acc, out, cid, sid)
    return k


def _sc_scatter(msg, idxp):
    return _sc_scatter_k()(msg, idxp)


@functools.cache
def _sc_count_k():
    @functools.partial(
        pl.kernel,
        out_type=jax.ShapeDtypeStruct((NC, N, F), jnp.float32),
        mesh=_sc_mesh(),
        compiler_params=pltpu.CompilerParams(use_tc_tiling_on_sc=False),
        scratch_types=[
            pltpu.VMEM((NCHUNK, CHP), jnp.int32),
            pltpu.VMEM((CHP, F), jnp.float32),
            pltpu.VMEM((NPS, F), jnp.float32),
            pltpu.VMEM_SHARED((NP, F), jnp.float32),
            pltpu.SemaphoreType.DMA,
        ],
    )
    def k(idx, out, idx_v, ones_v, zbuf, acc, sem):
        cid = lax.axis_index("c")
        sid = lax.axis_index("s")
        wid = sid * NC + cid
        pltpu.sync_copy(idx.at[pl.ds(wid * NCHUNK, NCHUNK)], idx_v)

        def orow(i, c):
            ones_v[i, :] = jnp.ones((L,), jnp.float32)
            return c
        lax.fori_loop(0, CHP, orow, 0)
        _zero_acc(zbuf, acc, sid)

        def body(j, c):
            pltpu.sync_copy(ones_v, acc.at[idx_v.at[j]], add=True)
            return c
        lax.fori_loop(0, NCHUNK, body, 0)
        _writeout_acc(acc, out, cid, sid)
    return k


def _sc_count(idxp):
    return _sc_count_k()(idxp)


def _leaky(x):
    return jnp.where(x > 0, x, 0.01 * x)


def _edge_math(ea, xg, w1, b1, w2, b2):
    h = jnp.dot(ea, w1, preferred_element_type=jnp.float32) + b1
    h = _leaky(h)
    h = jnp.dot(h, w2, preferred_element_type=jnp.float32) + b2
    acc = h[:, 0:F] * xg[:, 0:1]
    for i in range(1, F):
        acc = acc + h[:, i * F:(i + 1) * F] * xg[:, i:i + 1]
    return acc


def _edge_math_t(eaT, xgT, w1T, b1T, w2T, b2T):
    h = jnp.dot(w1T, eaT, preferred_element_type=jnp.float32) + b1T
    h = _leaky(h)
    h = jnp.dot(w2T, h, preferred_element_type=jnp.float32) + b2T
    acc = h[0:F, :] * xgT[0:1, :]
    for i in range(1, F):
        acc = acc + h[i * F:(i + 1) * F, :] * xgT[i:i + 1, :]
    return acc


def _bn_rows(u, g, b):
    m = jnp.mean(u, axis=0, keepdims=True)
    v = jnp.mean((u - m) ** 2, axis=0, keepdims=True)
    return (u - m) / jnp.sqrt(v + 1e-5) * g + b


def _node_math(x, sums, cnt16, root, bias, g, b):
    s = sums[0] + sums[1]
    c = cnt16[0] + cnt16[1]
    agg = s / jnp.maximum(c, 1.0)
    out = jnp.dot(x, root, preferred_element_type=jnp.float32) + agg + bias
    return _bn_rows(out, g, b)


def _final_math(h1, sums, cnt16, root, bias, g, b, mask, batch, te,
                w1a, w1b, b1, mg, mb, w2, b2):
    h = _node_math(h1, sums, cnt16, root, bias, g, b)
    gi = lax.broadcasted_iota(jnp.int32, (G, N), 0)
    bm = jnp.broadcast_to(batch, (G, N))
    mm = jnp.broadcast_to(mask, (G, N))
    p = jnp.where(gi == bm, mm, 0.0)
    pooled_s = jnp.dot(p, h, preferred_element_type=jnp.float32)
    cg = jnp.sum(p, axis=1, keepdims=True)
    pooled = pooled_s / jnp.maximum(cg, 1.0)
    u = (jnp.dot(pooled, w1a, preferred_element_type=jnp.float32)
         + jnp.dot(te, w1b, preferred_element_type=jnp.float32) + b1)
    u = _leaky(_bn_rows(u, mg, mb))
    return jnp.dot(u, w2, preferred_element_type=jnp.float32) + b2


def _edge_body(eaT_ref, xgT_ref, w1T_ref, b1T_ref, w2T_ref, b2T_ref,
               out_ref):
    out_ref[...] = _edge_math_t(eaT_ref[...], xgT_ref[...], w1T_ref[...],
                                b1T_ref[...], w2T_ref[...], b2T_ref[...])


def _edge_mlp(eaT, xgT, w1, b1, w2, b2):
    return pl.pallas_call(
        _edge_body,
        grid=(EP // BE,),
        in_specs=[
            pl.BlockSpec((ED, BE), lambda i: (0, i)),
            pl.BlockSpec((F, BE), lambda i: (0, i)),
            pl.BlockSpec((EW, ED), lambda i: (0, 0)),
            pl.BlockSpec((EW, 1), lambda i: (0, 0)),
            pl.BlockSpec((EW, EW), lambda i: (0, 0)),
            pl.BlockSpec((EW, 1), lambda i: (0, 0)),
        ],
        out_specs=pl.BlockSpec((F, BE), lambda i: (0, i)),
        out_shape=jax.ShapeDtypeStruct((F, EP), jnp.float32),
    )(eaT, xgT, w1.T, b1.reshape(EW, 1), w2.T, b2.reshape(EW, 1))


def _node_body(x_ref, s_ref, c_ref, root_ref, bias_ref, g_ref, b_ref, o_ref):
    o_ref[...] = _leaky(_node_math(
        x_ref[...], s_ref[...], c_ref[...], root_ref[...], bias_ref[...],
        g_ref[...], b_ref[...]))


def _node_update(x, sums, cnt16, root, bias, g, b):
    return pl.pallas_call(
        _node_body,
        out_shape=jax.ShapeDtypeStruct((N, F), jnp.float32),
    )(x, sums, cnt16, root, bias.reshape(1, F), g.reshape(1, F),
      b.reshape(1, F))


def _final_body(h1_ref, s_ref, c_ref, root_ref, bias_ref, g_ref, b_ref,
                mask_ref, batch_ref, te_ref, w1a_ref, w1b_ref, b1_ref,
                mg_ref, mb_ref, w2_ref, b2_ref, o_ref):
    o_ref[...] = _final_math(
        h1_ref[...], s_ref[...], c_ref[...], root_ref[...], bias_ref[...],
        g_ref[...], b_ref[...], mask_ref[...], batch_ref[...], te_ref[...],
        w1a_ref[...], w1b_ref[...], b1_ref[...], mg_ref[...], mb_ref[...],
        w2_ref[...], b2_ref[...])


def _final(h1, sums, cnt16, root, bias, g, b, mask, batch, te,
           w1a, w1b, b1, mg, mb, w2, b2):
    return pl.pallas_call(
        _final_body,
        out_shape=jax.ShapeDtypeStruct((G, 1), jnp.float32),
    )(h1, sums, cnt16, root, bias.reshape(1, F), g.reshape(1, F),
      b.reshape(1, F), mask, batch, te, w1a, w1b, b1.reshape(1, -1),
      mg.reshape(1, -1), mb.reshape(1, -1), w2, b2.reshape(1, 1))


def kernel(x, edge_index, edge_attr, trip_mask, batch, time_emb, num_graphs,
           root1, bias1, e1_w1, e1_b1, e1_w2, e1_b2, bn1_g, bn1_b,
           root2, bias2, e2_w1, e2_b1, e2_w2, e2_b2, bn2_g, bn2_b,
           m_w1, m_b1, m_bn_g, m_bn_b, m_w2, m_b2):
    src = edge_index[0]
    dst = edge_index[1]
    srcp = jnp.pad(src.reshape(E // CH, CH), ((0, 0), (0, CHP - CH)))
    dstp = jnp.pad(dst.reshape(E // CH, CH), ((0, 0), (0, CHP - CH)),
                   constant_values=N)
    eaT = jnp.pad(edge_attr.reshape(E // CH, CH, ED),
                  ((0, 0), (0, CHP - CH), (0, 0))).reshape(EP, ED).T

    xg1, cnt16 = _sc_gather_count(x, srcp, dstp)
    msg1 = _edge_mlp(eaT, xg1.T, e1_w1, e1_b1, e1_w2, e1_b2).T
    sums1 = _sc_scatter(msg1, dstp)
    h1 = _node_update(x, sums1, cnt16, root1, bias1, bn1_g, bn1_b)
    xg2 = _sc_gather(h1, srcp)
    msg2 = _edge_mlp(eaT, xg2.T, e2_w1, e2_b1, e2_w2, e2_b2).T
    sums2 = _sc_scatter(msg2, dstp)
    return _final(h1, sums2, cnt16, root2, bias2, bn2_g, bn2_b,
                  trip_mask.astype(jnp.float32).reshape(1, N),
                  batch.reshape(1, N), time_emb,
                  m_w1[:F], m_w1[F:], m_b1, m_bn_g, m_bn_b,
                  m_w2, m_b2)


# speedup vs baseline: 1.8292x; 1.8292x over previous
import functools

import jax
import jax.numpy as jnp
from jax import lax
from jax.experimental import pallas as pl
from jax.experimental.pallas import tpu as pltpu
from jax.experimental.pallas import tpu_sc as plsc

N = 10000
E = 160000
F = 16
ED = 4
EW = 256
G = 64

NC = 2
NS = 16
L = 16
NW = NC * NS
CH = 125
CHP = 128
NCHUNK = (E // NW) // CH
WPE = NCHUNK * CHP
EP = NW * WPE
NP = N + L
NPS = NP // NS
NWS = N // NS
BE = 4096

@functools.cache
def _sc_mesh():
    return plsc.VectorSubcoreMesh(
        core_axis_name="c", subcore_axis_name="s",
        num_cores=NC, num_subcores=NS)


def _zero_acc(zbuf, acc, sid):
    def zrow(i, c):
        zbuf[i, :] = jnp.zeros((L,), jnp.float32)
        return c
    lax.fori_loop(0, NPS, zrow, 0)
    pltpu.sync_copy(zbuf, acc.at[pl.ds(sid * NPS, NPS)])
    plsc.subcore_barrier()


def _writeout_acc(acc, out, cid, sid):
    plsc.subcore_barrier()
    @pl.when(sid == 0)
    def _():
        pltpu.sync_copy(acc.at[pl.ds(0, N)], out.at[cid])


@functools.cache
def _sc_gather_k():
    @functools.partial(
        pl.kernel,
        out_type=jax.ShapeDtypeStruct((EP, F), jnp.float32),
        mesh=_sc_mesh(),
        compiler_params=pltpu.CompilerParams(use_tc_tiling_on_sc=False),
        scratch_types=[
            pltpu.VMEM((NCHUNK, CHP), jnp.int32),
            pltpu.VMEM((WPE, F), jnp.float32),
            pltpu.VMEM_SHARED((N, F), jnp.float32),
            pltpu.SemaphoreType.DMA,
        ],
    )
    def k(table, idx, out, idx_v, rows_v, tsh, sem):
        cid = lax.axis_index("c")
        sid = lax.axis_index("s")
        wid = sid * NC + cid
        pltpu.sync_copy(idx.at[pl.ds(wid * NCHUNK, NCHUNK)], idx_v)

        @pl.when(sid == 0)
        def _():
            pltpu.sync_copy(table, tsh)
        plsc.subcore_barrier()

        def fire(j, c):
            pltpu.make_async_copy(tsh.at[idx_v.at[j]],
                                  rows_v.at[pl.ds(j * CHP, CHP)], sem).start()
            return c
        lax.fori_loop(0, NCHUNK, fire, 0)

        def drain(j, c):
            pltpu.make_async_copy(tsh.at[idx_v.at[j]],
                                  rows_v.at[pl.ds(j * CHP, CHP)], sem).wait()
            return c
        lax.fori_loop(0, NCHUNK, drain, 0)
        pltpu.sync_copy(rows_v, out.at[pl.ds(wid * WPE, WPE)])
    return k


def _sc_gather(table, idxp):
    return _sc_gather_k()(table, idxp)


@functools.cache
def _sc_gather_count_k():
    @functools.partial(
        pl.kernel,
        out_type=(jax.ShapeDtypeStruct((EP, F), jnp.float32),
                  jax.ShapeDtypeStruct((NC, N, F), jnp.float32)),
        mesh=_sc_mesh(),
        compiler_params=pltpu.CompilerParams(use_tc_tiling_on_sc=False),
        scratch_types=[
            pltpu.VMEM((NCHUNK, CHP), jnp.int32),
            pltpu.VMEM((NCHUNK, CHP), jnp.int32),
            pltpu.VMEM((WPE, F), jnp.float32),
            pltpu.VMEM((CHP, F), jnp.float32),
            pltpu.VMEM((NPS, F), jnp.float32),
            pltpu.VMEM_SHARED((N, F), jnp.float32),
            pltpu.VMEM_SHARED((NP, F), jnp.float32),
            pltpu.SemaphoreType.DMA,
            pltpu.SemaphoreType.DMA,
        ],
    )
    def k(table, sidx, didx, out, cnt, sidx_v, didx_v, rows_v, ones_v, zbuf,
          tsh, acc, gsem, csem):
        cid = lax.axis_index("c")
        sid = lax.axis_index("s")
        wid = sid * NC + cid
        pltpu.sync_copy(sidx.at[pl.ds(wid * NCHUNK, NCHUNK)], sidx_v)
        pltpu.sync_copy(didx.at[pl.ds(wid * NCHUNK, NCHUNK)], didx_v)

        def orow(i, c):
            ones_v[i, :] = jnp.ones((L,), jnp.float32)
            return c
        lax.fori_loop(0, CHP, orow, 0)

        @pl.when(sid == 0)
        def _():
            pltpu.sync_copy(table, tsh)
        _zero_acc(zbuf, acc, sid)

        def gfire(j, c):
            pltpu.make_async_copy(tsh.at[sidx_v.at[j]],
                                  rows_v.at[pl.ds(j * CHP, CHP)], gsem).start()
            return c
        lax.fori_loop(0, NCHUNK, gfire, 0)

        def cfire(j, c):
            pltpu.async_copy(ones_v, acc.at[didx_v.at[j]], csem, add=True)
            return c
        lax.fori_loop(0, NCHUNK, cfire, 0)

        def gdrain(j, c):
            pltpu.make_async_copy(tsh.at[sidx_v.at[j]],
                                  rows_v.at[pl.ds(j * CHP, CHP)], gsem).wait()
            return c
        lax.fori_loop(0, NCHUNK, gdrain, 0)
        pltpu.sync_copy(rows_v, out.at[pl.ds(wid * WPE, WPE)])

        def cdrain(j, c):
            pltpu.make_async_copy(ones_v, acc.at[didx_v.at[j]], csem).wait()
            return c
        lax.fori_loop(0, NCHUNK, cdrain, 0)
        _writeout_acc(acc, cnt, cid, sid)
    return k


def _sc_gather_count(table, sidxp, didxp):
    return _sc_gather_count_k()(table, sidxp, didxp)


@functools.cache
def _sc_scatter_k():
    @functools.partial(
        pl.kernel,
        out_type=jax.ShapeDtypeStruct((NC, N, F), jnp.float32),
        mesh=_sc_mesh(),
        compiler_params=pltpu.CompilerParams(use_tc_tiling_on_sc=False),
        scratch_types=[
            pltpu.VMEM((NCHUNK, CHP), jnp.int32),
            pltpu.VMEM((WPE, F), jnp.float32),
            pltpu.VMEM((NPS, F), jnp.float32),
            pltpu.VMEM_SHARED((NP, F), jnp.float32),
            pltpu.SemaphoreType.DMA,
        ],
    )
    def k(msg, idx, out, idx_v, rows_v, zbuf, acc, sem):
        cid = lax.axis_index("c")
        sid = lax.axis_index("s")
        wid = sid * NC + cid
        pltpu.sync_copy(idx.at[pl.ds(wid * NCHUNK, NCHUNK)], idx_v)
        cp = pltpu.make_async_copy(msg.at[pl.ds(wid * WPE, WPE)], rows_v, sem)
        cp.start()
        _zero_acc(zbuf, acc, sid)
        cp.wait()

        def fire(j, c):
            pltpu.async_copy(rows_v.at[pl.ds(j * CHP, CHP)],
                             acc.at[idx_v.at[j]], sem, add=True)
            return c
        lax.fori_loop(0, NCHUNK, fire, 0)

        def drain(j, c):
            pltpu.make_async_copy(rows_v.at[pl.ds(j * CHP, CHP)],
                                  acc.at[idx_v.at[j]], sem).wait()
            return c
        lax.fori_loop(0, NCHUNK, drain, 0)
        _writeout_acc(acc, out, cid, sid)
    return k


def _sc_scatter(msg, idxp):
    return _sc_scatter_k()(msg, idxp)


@functools.cache
def _sc_count_k():
    @functools.partial(
        pl.kernel,
        out_type=jax.ShapeDtypeStruct((NC, N, F), jnp.float32),
        mesh=_sc_mesh(),
        compiler_params=pltpu.CompilerParams(use_tc_tiling_on_sc=False),
        scratch_types=[
            pltpu.VMEM((NCHUNK, CHP), jnp.int32),
            pltpu.VMEM((CHP, F), jnp.float32),
            pltpu.VMEM((NPS, F), jnp.float32),
            pltpu.VMEM_SHARED((NP, F), jnp.float32),
            pltpu.SemaphoreType.DMA,
        ],
    )
    def k(idx, out, idx_v, ones_v, zbuf, acc, sem):
        cid = lax.axis_index("c")
        sid = lax.axis_index("s")
        wid = sid * NC + cid
        pltpu.sync_copy(idx.at[pl.ds(wid * NCHUNK, NCHUNK)], idx_v)

        def orow(i, c):
            ones_v[i, :] = jnp.ones((L,), jnp.float32)
            return c
        lax.fori_loop(0, CHP, orow, 0)
        _zero_acc(zbuf, acc, sid)

        def body(j, c):
            pltpu.sync_copy(ones_v, acc.at[idx_v.at[j]], add=True)
            return c
        lax.fori_loop(0, NCHUNK, body, 0)
        _writeout_acc(acc, out, cid, sid)
    return k


def _sc_count(idxp):
    return _sc_count_k()(idxp)


def _leaky(x):
    return jnp.where(x > 0, x, 0.01 * x)


def _edge_math(ea, xg, w1, b1, w2, b2):
    h = jnp.dot(ea, w1, preferred_element_type=jnp.float32) + b1
    h = _leaky(h)
    h = jnp.dot(h, w2, preferred_element_type=jnp.float32) + b2
    acc = h[:, 0:F] * xg[:, 0:1]
    for i in range(1, F):
        acc = acc + h[:, i * F:(i + 1) * F] * xg[:, i:i + 1]
    return acc


GRP = BE // 8


def _edge_math_t(eaTp, xg128, w1T, b1T, w2T, b2T):
    h = jnp.dot(w1T, eaTp, preferred_element_type=jnp.float32) + b1T
    h = _leaky(h)
    h = jnp.dot(w2T, h, preferred_element_type=jnp.float32) + b2T
    blkT = jnp.transpose(xg128)
    pieces = []
    for p in range(8):
        acc = None
        for i in range(F):
            t = (h[i * F:(i + 1) * F, p * GRP:(p + 1) * GRP]
                 * blkT[16 * p + i:16 * p + i + 1, :])
            acc = t if acc is None else acc + t
        pieces.append(acc)
    out_pre = jnp.concatenate(pieces, axis=0)
    return jnp.transpose(out_pre)


def _bn_rows(u, g, b):
    m = jnp.mean(u, axis=0, keepdims=True)
    v = jnp.mean((u - m) ** 2, axis=0, keepdims=True)
    return (u - m) / jnp.sqrt(v + 1e-5) * g + b


def _node_math(x, sums, cnt16, root, bias, g, b):
    s = sums[0] + sums[1]
    c = cnt16[0] + cnt16[1]
    agg = s / jnp.maximum(c, 1.0)
    out = jnp.dot(x, root, preferred_element_type=jnp.float32) + agg + bias
    return _bn_rows(out, g, b)


def _final_math(h1, sums, cnt16, root, bias, g, b, mask, batch, te,
                w1a, w1b, b1, mg, mb, w2, b2):
    h = _node_math(h1, sums, cnt16, root, bias, g, b)
    gi = lax.broadcasted_iota(jnp.int32, (G, N), 0)
    bm = jnp.broadcast_to(batch, (G, N))
    mm = jnp.broadcast_to(mask, (G, N))
    p = jnp.where(gi == bm, mm, 0.0)
    pooled_s = jnp.dot(p, h, preferred_element_type=jnp.float32)
    cg = jnp.sum(p, axis=1, keepdims=True)
    pooled = pooled_s / jnp.maximum(cg, 1.0)
    u = (jnp.dot(pooled, w1a, preferred_element_type=jnp.float32)
         + jnp.dot(te, w1b, preferred_element_type=jnp.float32) + b1)
    u = _leaky(_bn_rows(u, mg, mb))
    return jnp.dot(u, w2, preferred_element_type=jnp.float32) + b2


def _edge_body(eaT_ref, xg_ref, w1T_ref, b1T_ref, w2T_ref, b2T_ref, out_ref):
    out_ref[...] = _edge_math_t(eaT_ref[...], xg_ref[...], w1T_ref[...],
                                b1T_ref[...], w2T_ref[...], b2T_ref[...])


def _edge_mlp(eaTp, xg128, w1, b1, w2, b2):
    return pl.pallas_call(
        _edge_body,
        grid=(EP // BE,),
        in_specs=[
            pl.BlockSpec((ED, BE), lambda i: (0, i)),
            pl.BlockSpec((BE // 8, 128), lambda i: (i, 0)),
            pl.BlockSpec((EW, ED), lambda i: (0, 0)),
            pl.BlockSpec((EW, 1), lambda i: (0, 0)),
            pl.BlockSpec((EW, EW), lambda i: (0, 0)),
            pl.BlockSpec((EW, 1), lambda i: (0, 0)),
        ],
        out_specs=pl.BlockSpec((BE // 8, 128), lambda i: (i, 0)),
        out_shape=jax.ShapeDtypeStruct((EP // 8, 128), jnp.float32),
    )(eaTp, xg128, w1.T, b1.reshape(EW, 1), w2.T, b2.reshape(EW, 1))


def _node_body(x_ref, s_ref, c_ref, root_ref, bias_ref, g_ref, b_ref, o_ref):
    o_ref[...] = _leaky(_node_math(
        x_ref[...], s_ref[...], c_ref[...], root_ref[...], bias_ref[...],
        g_ref[...], b_ref[...]))


def _node_update(x, sums, cnt16, root, bias, g, b):
    return pl.pallas_call(
        _node_body,
        out_shape=jax.ShapeDtypeStruct((N, F), jnp.float32),
    )(x, sums, cnt16, root, bias.reshape(1, F), g.reshape(1, F),
      b.reshape(1, F))


def _final_body(h1_ref, s_ref, c_ref, root_ref, bias_ref, g_ref, b_ref,
                mask_ref, batch_ref, te_ref, w1a_ref, w1b_ref, b1_ref,
                mg_ref, mb_ref, w2_ref, b2_ref, o_ref):
    o_ref[...] = _final_math(
        h1_ref[...], s_ref[...], c_ref[...], root_ref[...], bias_ref[...],
        g_ref[...], b_ref[...], mask_ref[...], batch_ref[...], te_ref[...],
        w1a_ref[...], w1b_ref[...], b1_ref[...], mg_ref[...], mb_ref[...],
        w2_ref[...], b2_ref[...])


def _final(h1, sums, cnt16, root, bias, g, b, mask, batch, te,
           w1a, w1b, b1, mg, mb, w2, b2):
    return pl.pallas_call(
        _final_body,
        out_shape=jax.ShapeDtypeStruct((G, 1), jnp.float32),
    )(h1, sums, cnt16, root, bias.reshape(1, F), g.reshape(1, F),
      b.reshape(1, F), mask, batch, te, w1a, w1b, b1.reshape(1, -1),
      mg.reshape(1, -1), mb.reshape(1, -1), w2, b2.reshape(1, 1))


def kernel(x, edge_index, edge_attr, trip_mask, batch, time_emb, num_graphs,
           root1, bias1, e1_w1, e1_b1, e1_w2, e1_b2, bn1_g, bn1_b,
           root2, bias2, e2_w1, e2_b1, e2_w2, e2_b2, bn2_g, bn2_b,
           m_w1, m_b1, m_bn_g, m_bn_b, m_w2, m_b2):
    src = edge_index[0]
    dst = edge_index[1]
    srcp = jnp.pad(src.reshape(E // CH, CH), ((0, 0), (0, CHP - CH)))
    dstp = jnp.pad(dst.reshape(E // CH, CH), ((0, 0), (0, CHP - CH)),
                   constant_values=N)
    eaT = jnp.pad(edge_attr.reshape(E // CH, CH, ED),
                  ((0, 0), (0, CHP - CH), (0, 0))).reshape(EP, ED).T
    eaTp = (eaT.reshape(ED, EP // BE, GRP, 8)
            .transpose(0, 1, 3, 2).reshape(ED, EP))

    xg1, cnt16 = _sc_gather_count(x, srcp, dstp)
    msg1 = _edge_mlp(eaTp, xg1.reshape(EP // 8, 128),
                     e1_w1, e1_b1, e1_w2, e1_b2).reshape(EP, F)
    sums1 = _sc_scatter(msg1, dstp)
    h1 = _node_update(x, sums1, cnt16, root1, bias1, bn1_g, bn1_b)
    xg2 = _sc_gather(h1, srcp)
    msg2 = _edge_mlp(eaTp, xg2.reshape(EP // 8, 128),
                     e2_w1, e2_b1, e2_w2, e2_b2).reshape(EP, F)
    sums2 = _sc_scatter(msg2, dstp)
    return _final(h1, sums2, cnt16, root2, bias2, bn2_g, bn2_b,
                  trip_mask.astype(jnp.float32).reshape(1, N),
                  batch.reshape(1, N), time_emb,
                  m_w1[:F], m_w1[F:], m_b1, m_bn_g, m_bn_b,
                  m_w2, m_b2)


# speedup vs baseline: 2.0388x; 1.1146x over previous
import functools

import jax
import jax.numpy as jnp
from jax import lax
from jax.experimental import pallas as pl
from jax.experimental.pallas import tpu as pltpu
from jax.experimental.pallas import tpu_sc as plsc

N = 10000
E = 160000
F = 16
ED = 4
EW = 256
G = 64

NC = 2
NS = 16
L = 16
NW = NC * NS
CH = 125
CHP = 128
NCHUNK = (E // NW) // CH
WPE = NCHUNK * CHP
EP = NW * WPE
NP = N + L
NPS = NP // NS
NWS = N // NS
BE = 4096

@functools.cache
def _sc_mesh():
    return plsc.VectorSubcoreMesh(
        core_axis_name="c", subcore_axis_name="s",
        num_cores=NC, num_subcores=NS)


def _zero_acc(zbuf, acc, sid):
    def zrow(i, c):
        zbuf[i, :] = jnp.zeros((L,), jnp.float32)
        return c
    lax.fori_loop(0, NPS, zrow, 0)
    pltpu.sync_copy(zbuf, acc.at[pl.ds(sid * NPS, NPS)])
    plsc.subcore_barrier()


def _writeout_acc(acc, out, cid, sid):
    plsc.subcore_barrier()
    @pl.when(sid == 0)
    def _():
        pltpu.sync_copy(acc.at[pl.ds(0, N)], out.at[cid])


@functools.cache
def _sc_gather_k():
    @functools.partial(
        pl.kernel,
        out_type=jax.ShapeDtypeStruct((EP, F), jnp.float32),
        mesh=_sc_mesh(),
        compiler_params=pltpu.CompilerParams(use_tc_tiling_on_sc=False),
        scratch_types=[
            pltpu.VMEM((NCHUNK, CHP), jnp.int32),
            pltpu.VMEM((WPE, F), jnp.float32),
            pltpu.VMEM_SHARED((N, F), jnp.float32),
            pltpu.SemaphoreType.DMA,
        ],
    )
    def k(table, idx, out, idx_v, rows_v, tsh, sem):
        cid = lax.axis_index("c")
        sid = lax.axis_index("s")
        wid = sid * NC + cid
        pltpu.sync_copy(idx.at[pl.ds(wid * NCHUNK, NCHUNK)], idx_v)

        @pl.when(sid == 0)
        def _():
            pltpu.sync_copy(table, tsh)
        plsc.subcore_barrier()

        def fire(j, c):
            pltpu.make_async_copy(tsh.at[idx_v.at[j]],
                                  rows_v.at[pl.ds(j * CHP, CHP)], sem).start()
            return c
        lax.fori_loop(0, NCHUNK, fire, 0)

        def drain(j, c):
            pltpu.make_async_copy(tsh.at[idx_v.at[j]],
                                  rows_v.at[pl.ds(j * CHP, CHP)], sem).wait()
            return c
        lax.fori_loop(0, NCHUNK, drain, 0)
        pltpu.sync_copy(rows_v, out.at[pl.ds(wid * WPE, WPE)])
    return k


def _sc_gather(table, idxp):
    return _sc_gather_k()(table, idxp)


@functools.cache
def _sc_gather_count_k():
    @functools.partial(
        pl.kernel,
        out_type=(jax.ShapeDtypeStruct((EP, F), jnp.float32),
                  jax.ShapeDtypeStruct((NC, N, F), jnp.float32)),
        mesh=_sc_mesh(),
        compiler_params=pltpu.CompilerParams(use_tc_tiling_on_sc=False),
        scratch_types=[
            pltpu.VMEM((NCHUNK, CHP), jnp.int32),
            pltpu.VMEM((NCHUNK, CHP), jnp.int32),
            pltpu.VMEM((WPE, F), jnp.float32),
            pltpu.VMEM((CHP, F), jnp.float32),
            pltpu.VMEM((NPS, F), jnp.float32),
            pltpu.VMEM_SHARED((N, F), jnp.float32),
            pltpu.VMEM_SHARED((NP, F), jnp.float32),
            pltpu.SemaphoreType.DMA,
            pltpu.SemaphoreType.DMA,
        ],
    )
    def k(table, sidx, didx, out, cnt, sidx_v, didx_v, rows_v, ones_v, zbuf,
          tsh, acc, gsem, csem):
        cid = lax.axis_index("c")
        sid = lax.axis_index("s")
        wid = sid * NC + cid
        pltpu.sync_copy(sidx.at[pl.ds(wid * NCHUNK, NCHUNK)], sidx_v)
        pltpu.sync_copy(didx.at[pl.ds(wid * NCHUNK, NCHUNK)], didx_v)

        def orow(i, c):
            ones_v[i, :] = jnp.ones((L,), jnp.float32)
            return c
        lax.fori_loop(0, CHP, orow, 0)

        @pl.when(sid == 0)
        def _():
            pltpu.sync_copy(table, tsh)
        _zero_acc(zbuf, acc, sid)

        def gfire(j, c):
            pltpu.make_async_copy(tsh.at[sidx_v.at[j]],
                                  rows_v.at[pl.ds(j * CHP, CHP)], gsem).start()
            return c
        lax.fori_loop(0, NCHUNK, gfire, 0)

        def cfire(j, c):
            pltpu.async_copy(ones_v, acc.at[didx_v.at[j]], csem, add=True)
            return c
        lax.fori_loop(0, NCHUNK, cfire, 0)

        def gdrain(j, c):
            pltpu.make_async_copy(tsh.at[sidx_v.at[j]],
                                  rows_v.at[pl.ds(j * CHP, CHP)], gsem).wait()
            return c
        lax.fori_loop(0, NCHUNK, gdrain, 0)
        pltpu.sync_copy(rows_v, out.at[pl.ds(wid * WPE, WPE)])

        def cdrain(j, c):
            pltpu.make_async_copy(ones_v, acc.at[didx_v.at[j]], csem).wait()
            return c
        lax.fori_loop(0, NCHUNK, cdrain, 0)
        _writeout_acc(acc, cnt, cid, sid)
    return k


def _sc_gather_count(table, sidxp, didxp):
    return _sc_gather_count_k()(table, sidxp, didxp)


@functools.cache
def _sc_scatter_k():
    @functools.partial(
        pl.kernel,
        out_type=jax.ShapeDtypeStruct((NC, N, F), jnp.float32),
        mesh=_sc_mesh(),
        compiler_params=pltpu.CompilerParams(use_tc_tiling_on_sc=False),
        scratch_types=[
            pltpu.VMEM((NCHUNK, CHP), jnp.int32),
            pltpu.VMEM((WPE, F), jnp.float32),
            pltpu.VMEM((NPS, F), jnp.float32),
            pltpu.VMEM_SHARED((NP, F), jnp.float32),
            pltpu.SemaphoreType.DMA,
        ],
    )
    def k(msg, idx, out, idx_v, rows_v, zbuf, acc, sem):
        cid = lax.axis_index("c")
        sid = lax.axis_index("s")
        wid = sid * NC + cid
        pltpu.sync_copy(idx.at[pl.ds(wid * NCHUNK, NCHUNK)], idx_v)
        cp = pltpu.make_async_copy(msg.at[pl.ds(wid * WPE, WPE)], rows_v, sem)
        cp.start()
        _zero_acc(zbuf, acc, sid)
        cp.wait()

        def fire(j, c):
            pltpu.async_copy(rows_v.at[pl.ds(j * CHP, CHP)],
                             acc.at[idx_v.at[j]], sem, add=True)
            return c
        lax.fori_loop(0, NCHUNK, fire, 0)

        def drain(j, c):
            pltpu.make_async_copy(rows_v.at[pl.ds(j * CHP, CHP)],
                                  acc.at[idx_v.at[j]], sem).wait()
            return c
        lax.fori_loop(0, NCHUNK, drain, 0)
        _writeout_acc(acc, out, cid, sid)
    return k


def _sc_scatter(msg, idxp):
    return _sc_scatter_k()(msg, idxp)


@functools.cache
def _sc_count_k():
    @functools.partial(
        pl.kernel,
        out_type=jax.ShapeDtypeStruct((NC, N, F), jnp.float32),
        mesh=_sc_mesh(),
        compiler_params=pltpu.CompilerParams(use_tc_tiling_on_sc=False),
        scratch_types=[
            pltpu.VMEM((NCHUNK, CHP), jnp.int32),
            pltpu.VMEM((CHP, F), jnp.float32),
            pltpu.VMEM((NPS, F), jnp.float32),
            pltpu.VMEM_SHARED((NP, F), jnp.float32),
            pltpu.SemaphoreType.DMA,
        ],
    )
    def k(idx, out, idx_v, ones_v, zbuf, acc, sem):
        cid = lax.axis_index("c")
        sid = lax.axis_index("s")
        wid = sid * NC + cid
        pltpu.sync_copy(idx.at[pl.ds(wid * NCHUNK, NCHUNK)], idx_v)

        def orow(i, c):
            ones_v[i, :] = jnp.ones((L,), jnp.float32)
            return c
        lax.fori_loop(0, CHP, orow, 0)
        _zero_acc(zbuf, acc, sid)

        def body(j, c):
            pltpu.sync_copy(ones_v, acc.at[idx_v.at[j]], add=True)
            return c
        lax.fori_loop(0, NCHUNK, body, 0)
        _writeout_acc(acc, out, cid, sid)
    return k


def _sc_count(idxp):
    return _sc_count_k()(idxp)


def _leaky(x):
    return jnp.where(x > 0, x, 0.01 * x)


def _edge_math(ea, xg, w1, b1, w2, b2):
    h = jnp.dot(ea, w1, preferred_element_type=jnp.float32) + b1
    h = _leaky(h)
    h = jnp.dot(h, w2, preferred_element_type=jnp.float32) + b2
    acc = h[:, 0:F] * xg[:, 0:1]
    for i in range(1, F):
        acc = acc + h[:, i * F:(i + 1) * F] * xg[:, i:i + 1]
    return acc


GRP = BE // 8


def _edge_math_t(eaTp, xg128, w1T, b1T, w2T, b2T):
    h = jnp.dot(w1T, eaTp, preferred_element_type=jnp.float32) + b1T
    h = _leaky(h)
    h = jnp.dot(w2T, h.astype(jnp.bfloat16),
                preferred_element_type=jnp.float32) + b2T
    blkT = jnp.transpose(xg128)
    pieces = []
    for p in range(8):
        acc = None
        for i in range(F):
            t = (h[i * F:(i + 1) * F, p * GRP:(p + 1) * GRP]
                 * blkT[16 * p + i:16 * p + i + 1, :])
            acc = t if acc is None else acc + t
        pieces.append(acc)
    out_pre = jnp.concatenate(pieces, axis=0)
    return jnp.transpose(out_pre)


def _bn_rows(u, g, b):
    m = jnp.mean(u, axis=0, keepdims=True)
    v = jnp.mean((u - m) ** 2, axis=0, keepdims=True)
    return (u - m) / jnp.sqrt(v + 1e-5) * g + b


def _node_math(x, sums, cnt16, root, bias, g, b):
    s = sums[0] + sums[1]
    c = cnt16[0] + cnt16[1]
    agg = s / jnp.maximum(c, 1.0)
    out = jnp.dot(x, root, preferred_element_type=jnp.float32) + agg + bias
    return _bn_rows(out, g, b)


def _final_math(h1, sums, cnt16, root, bias, g, b, mask, batch, te,
                w1a, w1b, b1, mg, mb, w2, b2):
    h = _node_math(h1, sums, cnt16, root, bias, g, b)
    gi = lax.broadcasted_iota(jnp.int32, (G, N), 0)
    bm = jnp.broadcast_to(batch, (G, N))
    mm = jnp.broadcast_to(mask, (G, N))
    p = jnp.where(gi == bm, mm, 0.0)
    pooled_s = jnp.dot(p, h, preferred_element_type=jnp.float32)
    cg = jnp.sum(p, axis=1, keepdims=True)
    pooled = pooled_s / jnp.maximum(cg, 1.0)
    u = (jnp.dot(pooled, w1a, preferred_element_type=jnp.float32)
         + jnp.dot(te, w1b, preferred_element_type=jnp.float32) + b1)
    u = _leaky(_bn_rows(u, mg, mb))
    return jnp.dot(u, w2, preferred_element_type=jnp.float32) + b2


def _edge_body(eaT_ref, xg_ref, w1T_ref, b1T_ref, w2T_ref, b2T_ref, out_ref):
    out_ref[...] = _edge_math_t(eaT_ref[...], xg_ref[...], w1T_ref[...],
                                b1T_ref[...], w2T_ref[...], b2T_ref[...])


def _edge_mlp(eaTp, xg128, w1, b1, w2, b2):
    return pl.pallas_call(
        _edge_body,
        grid=(EP // BE,),
        in_specs=[
            pl.BlockSpec((ED, BE), lambda i: (0, i)),
            pl.BlockSpec((BE // 8, 128), lambda i: (i, 0)),
            pl.BlockSpec((EW, ED), lambda i: (0, 0)),
            pl.BlockSpec((EW, 1), lambda i: (0, 0)),
            pl.BlockSpec((EW, EW), lambda i: (0, 0)),
            pl.BlockSpec((EW, 1), lambda i: (0, 0)),
        ],
        out_specs=pl.BlockSpec((BE // 8, 128), lambda i: (i, 0)),
        out_shape=jax.ShapeDtypeStruct((EP // 8, 128), jnp.float32),
    )(eaTp, xg128, w1.T, b1.reshape(EW, 1),
      w2.T.astype(jnp.bfloat16), b2.reshape(EW, 1))


def _node_body(x_ref, s_ref, c_ref, root_ref, bias_ref, g_ref, b_ref, o_ref):
    o_ref[...] = _leaky(_node_math(
        x_ref[...], s_ref[...], c_ref[...], root_ref[...], bias_ref[...],
        g_ref[...], b_ref[...]))


def _node_update(x, sums, cnt16, root, bias, g, b):
    return pl.pallas_call(
        _node_body,
        out_shape=jax.ShapeDtypeStruct((N, F), jnp.float32),
    )(x, sums, cnt16, root, bias.reshape(1, F), g.reshape(1, F),
      b.reshape(1, F))


def _final_body(h1_ref, s_ref, c_ref, root_ref, bias_ref, g_ref, b_ref,
                mask_ref, batch_ref, te_ref, w1a_ref, w1b_ref, b1_ref,
                mg_ref, mb_ref, w2_ref, b2_ref, o_ref):
    o_ref[...] = _final_math(
        h1_ref[...], s_ref[...], c_ref[...], root_ref[...], bias_ref[...],
        g_ref[...], b_ref[...], mask_ref[...], batch_ref[...], te_ref[...],
        w1a_ref[...], w1b_ref[...], b1_ref[...], mg_ref[...], mb_ref[...],
        w2_ref[...], b2_ref[...])


def _final(h1, sums, cnt16, root, bias, g, b, mask, batch, te,
           w1a, w1b, b1, mg, mb, w2, b2):
    return pl.pallas_call(
        _final_body,
        out_shape=jax.ShapeDtypeStruct((G, 1), jnp.float32),
    )(h1, sums, cnt16, root, bias.reshape(1, F), g.reshape(1, F),
      b.reshape(1, F), mask, batch, te, w1a, w1b, b1.reshape(1, -1),
      mg.reshape(1, -1), mb.reshape(1, -1), w2, b2.reshape(1, 1))


def kernel(x, edge_index, edge_attr, trip_mask, batch, time_emb, num_graphs,
           root1, bias1, e1_w1, e1_b1, e1_w2, e1_b2, bn1_g, bn1_b,
           root2, bias2, e2_w1, e2_b1, e2_w2, e2_b2, bn2_g, bn2_b,
           m_w1, m_b1, m_bn_g, m_bn_b, m_w2, m_b2):
    src = edge_index[0]
    dst = edge_index[1]
    srcp = jnp.pad(src.reshape(E // CH, CH), ((0, 0), (0, CHP - CH)))
    dstp = jnp.pad(dst.reshape(E // CH, CH), ((0, 0), (0, CHP - CH)),
                   constant_values=N)
    eaT = jnp.pad(edge_attr.reshape(E // CH, CH, ED),
                  ((0, 0), (0, CHP - CH), (0, 0))).reshape(EP, ED).T
    srcp = (srcp.reshape(EP // BE, 8, GRP)
            .transpose(0, 2, 1).reshape(NW * NCHUNK, CHP))
    dstp = (dstp.reshape(EP // BE, 8, GRP)
            .transpose(0, 2, 1).reshape(NW * NCHUNK, CHP))

    xg1, cnt16 = _sc_gather_count(x, srcp, dstp)
    msg1 = _edge_mlp(eaT, xg1.reshape(EP // 8, 128),
                     e1_w1, e1_b1, e1_w2, e1_b2).reshape(EP, F)
    sums1 = _sc_scatter(msg1, dstp)
    h1 = _node_update(x, sums1, cnt16, root1, bias1, bn1_g, bn1_b)
    xg2 = _sc_gather(h1, srcp)
    msg2 = _edge_mlp(eaT, xg2.reshape(EP // 8, 128),
                     e2_w1, e2_b1, e2_w2, e2_b2).reshape(EP, F)
    sums2 = _sc_scatter(msg2, dstp)
    return _final(h1, sums2, cnt16, root2, bias2, bn2_g, bn2_b,
                  trip_mask.astype(jnp.float32).reshape(1, N),
                  batch.reshape(1, N), time_emb,
                  m_w1[:F], m_w1[F:], m_b1, m_bn_g, m_bn_b,
                  m_w2, m_b2)
